# baseline (speedup 1.0000x reference)
import functools

import jax
import jax.numpy as jnp
from jax import lax
from jax.experimental import pallas as pl
from jax.experimental.pallas import tpu as pltpu
from jax.experimental.pallas import tpu_sc as plsc

VOCAB = 1000
VPAD = 1024
N_TOK = 16 * 2048

_info = plsc.get_sparse_core_info()
NC, NS, L = _info.num_cores, _info.num_subcores, _info.num_lanes
NW = NC * NS
TOK_PER_W = N_TOK // NW
CHUNK = 32
N_CHUNK = TOK_PER_W // CHUNK
GRP = CHUNK // L


def _lse_body(emb_ref, lse_ref, emb_pad_ref):
    x = emb_ref[...]
    m = jnp.max(x, axis=1)
    s = jnp.sum(jnp.exp(x - m[:, None]), axis=1)
    lse_ref[...] = m + jnp.log(s)
    emb_pad_ref[...] = jnp.concatenate(
        [x, jnp.zeros((VOCAB, VPAD - VOCAB), jnp.float32)], axis=1
    )


def _row_lse(emb):
    return pl.pallas_call(
        _lse_body,
        out_shape=[
            jax.ShapeDtypeStruct((VOCAB,), jnp.float32),
            jax.ShapeDtypeStruct((VOCAB, VPAD), jnp.float32),
        ],
    )(emb)


def _sc_gather(idx_flat, tgt_flat, emb_pad, lse):
    mesh = plsc.VectorSubcoreMesh(core_axis_name="c", subcore_axis_name="s")

    @functools.partial(
        pl.kernel,
        mesh=mesh,
        compiler_params=pltpu.CompilerParams(needs_layout_passes=False),
        out_type=[
            jax.ShapeDtypeStruct((16, 2048, VOCAB), jnp.float32),
            jax.ShapeDtypeStruct((N_TOK, 128), jnp.float32),
            jax.ShapeDtypeStruct((NW * 128,), jnp.float32),
        ],
        scratch_types=[
            pltpu.VMEM((TOK_PER_W,), jnp.int32),
            pltpu.VMEM((TOK_PER_W,), jnp.int32),
            pltpu.VMEM((VOCAB,), jnp.float32),
            pltpu.VMEM((CHUNK, VPAD), jnp.float32),
            pltpu.VMEM((CHUNK, VPAD), jnp.float32),
            pltpu.VMEM((L,), jnp.float32),
            pltpu.SemaphoreType.DMA,
            pltpu.SemaphoreType.DMA,
        ],
    )
    def k(idx_hbm, tgt_hbm, emb_hbm, lse_hbm, out_hbm, tail_hbm, part_hbm,
          idx_v, tgt_v, lse_v, rows_a, rows_b, acc_v, sem_g, sem_s):
        wid = lax.axis_index("s") * NC + lax.axis_index("c")
        base = wid * TOK_PER_W
        b_idx = wid // 2
        t_base = (wid % 2) * TOK_PER_W
        pltpu.sync_copy(idx_hbm.at[pl.ds(base, TOK_PER_W)], idx_v)
        pltpu.sync_copy(tgt_hbm.at[pl.ds(base, TOK_PER_W)], tgt_v)
        pltpu.sync_copy(lse_hbm, lse_v)
        acc_v[...] = jnp.zeros((L,), jnp.float32)

        bufs = [rows_a, rows_b]

        def gather(c, buf):
            return pltpu.async_copy(
                emb_hbm.at[idx_v.at[pl.ds(c * CHUNK, CHUNK)]], buf, sem_g
            )

        def compute(c):
            for g in range(GRP):
                t0 = c * CHUNK + g * L
                iv = idx_v[pl.ds(t0, L)]
                tv = tgt_v[pl.ds(t0, L)]
                lse16 = plsc.load_gather(lse_v, [iv])
                rid = lax.iota(jnp.int32, L) + g * L
                tval = plsc.load_gather(bufs[c % 2], [rid, tv])
                acc_v[...] = acc_v[...] + (lse16 - tval)

        def scatter(c):
            buf = bufs[c % 2]
            d1 = pltpu.async_copy(
                buf.at[:, pl.ds(0, 896)],
                out_hbm.at[b_idx, pl.ds(t_base + c * CHUNK, CHUNK),
                           pl.ds(0, 896)],
                sem_s,
            )
            d2 = pltpu.async_copy(
                buf.at[:, pl.ds(896, 128)],
                tail_hbm.at[pl.ds(base + c * CHUNK, CHUNK)],
                sem_s,
            )
            return (d1, d2)

        g_desc = gather(0, bufs[0])
        s_desc = [None] * N_CHUNK
        for c in range(N_CHUNK):
            g_desc.wait()
            if c + 1 < N_CHUNK:
                if c >= 1:
                    s_desc[c - 1][0].wait()
                    s_desc[c - 1][1].wait()
                g_desc = gather(c + 1, bufs[(c + 1) % 2])
            compute(c)
            s_desc[c] = scatter(c)
        for c in (N_CHUNK - 2, N_CHUNK - 1):
            s_desc[c][0].wait()
            s_desc[c][1].wait()
        pltpu.sync_copy(acc_v, part_hbm.at[pl.ds(wid * 128, L)])

    return k(idx_flat, tgt_flat, emb_pad, lse)


def kernel(idx, targets, tok_emb):
    idx_flat = idx.reshape(-1).astype(jnp.int32)
    tgt_flat = targets.reshape(-1).astype(jnp.int32)
    lse, emb_pad = _row_lse(tok_emb)
    logits_main, tails, partials = _sc_gather(idx_flat, tgt_flat, emb_pad, lse)
    tail = tails[:, : VOCAB - 896].reshape(16, 2048, VOCAB - 896)
    logits = lax.dynamic_update_slice(logits_main, tail, (0, 0, 896))
    loss = jnp.sum(partials.reshape(NW, 128)[:, :L]) / N_TOK
    return (logits, loss)


# speedup vs baseline: 1.0505x; 1.0505x over previous
import functools

import jax
import jax.numpy as jnp
from jax import lax
from jax.experimental import pallas as pl
from jax.experimental.pallas import tpu as pltpu
from jax.experimental.pallas import tpu_sc as plsc

VOCAB = 1000
VPAD = 1024
N_TOK = 16 * 2048

_info = plsc.get_sparse_core_info()
NC, NS, L = _info.num_cores, _info.num_subcores, _info.num_lanes
NW = NC * NS
TOK_PER_W = N_TOK // NW
CHUNK = 32
N_CHUNK = TOK_PER_W // CHUNK
GRP = CHUNK // L


def _lse_body(emb_ref, lse_ref, emb_pad_ref):
    x = emb_ref[...]
    m = jnp.max(x, axis=1)
    s = jnp.sum(jnp.exp(x - m[:, None]), axis=1)
    lse_ref[...] = m + jnp.log(s)
    emb_pad_ref[...] = jnp.concatenate(
        [x, jnp.zeros((VOCAB, VPAD - VOCAB), jnp.float32)], axis=1
    )


def _row_lse(emb):
    return pl.pallas_call(
        _lse_body,
        out_shape=[
            jax.ShapeDtypeStruct((VOCAB,), jnp.float32),
            jax.ShapeDtypeStruct((VOCAB, VPAD), jnp.float32),
        ],
    )(emb)


def _sc_gather(idx_flat, tgt_flat, emb_pad, lse):
    mesh = plsc.VectorSubcoreMesh(core_axis_name="c", subcore_axis_name="s")

    @functools.partial(
        pl.kernel,
        mesh=mesh,
        compiler_params=pltpu.CompilerParams(needs_layout_passes=False),
        out_type=[
            jax.ShapeDtypeStruct((N_TOK, VPAD), jnp.float32),
            jax.ShapeDtypeStruct((NW * 128,), jnp.float32),
        ],
        scratch_types=[
            pltpu.VMEM((TOK_PER_W,), jnp.int32),
            pltpu.VMEM((TOK_PER_W,), jnp.int32),
            pltpu.VMEM((VOCAB,), jnp.float32),
            pltpu.VMEM((CHUNK, VPAD), jnp.float32),
            pltpu.VMEM((CHUNK, VPAD), jnp.float32),
            pltpu.VMEM((L,), jnp.float32),
            pltpu.SemaphoreType.DMA,
            pltpu.SemaphoreType.DMA,
        ],
    )
    def k(idx_hbm, tgt_hbm, emb_hbm, lse_hbm, out_hbm, part_hbm,
          idx_v, tgt_v, lse_v, rows_a, rows_b, acc_v, sem_g, sem_s):
        wid = lax.axis_index("s") * NC + lax.axis_index("c")
        base = wid * TOK_PER_W
        pltpu.sync_copy(idx_hbm.at[pl.ds(base, TOK_PER_W)], idx_v)
        pltpu.sync_copy(tgt_hbm.at[pl.ds(base, TOK_PER_W)], tgt_v)
        pltpu.sync_copy(lse_hbm, lse_v)
        acc_v[...] = jnp.zeros((L,), jnp.float32)

        bufs = [rows_a, rows_b]

        def gather(c, buf):
            return pltpu.async_copy(
                emb_hbm.at[idx_v.at[pl.ds(c * CHUNK, CHUNK)]], buf, sem_g
            )

        def compute(c):
            for g in range(GRP):
                t0 = c * CHUNK + g * L
                iv = idx_v[pl.ds(t0, L)]
                tv = tgt_v[pl.ds(t0, L)]
                lse16 = plsc.load_gather(lse_v, [iv])
                rid = lax.iota(jnp.int32, L) + g * L
                tval = plsc.load_gather(bufs[c % 2], [rid, tv])
                acc_v[...] = acc_v[...] + (lse16 - tval)

        def scatter(c):
            return pltpu.async_copy(
                bufs[c % 2], out_hbm.at[pl.ds(base + c * CHUNK, CHUNK)], sem_s
            )

        g_desc = gather(0, bufs[0])
        s_desc = [None] * N_CHUNK
        for c in range(N_CHUNK):
            g_desc.wait()
            if c + 1 < N_CHUNK:
                if c >= 1:
                    s_desc[c - 1].wait()
                g_desc = gather(c + 1, bufs[(c + 1) % 2])
            compute(c)
            s_desc[c] = scatter(c)
        s_desc[N_CHUNK - 2].wait()
        s_desc[N_CHUNK - 1].wait()
        pltpu.sync_copy(acc_v, part_hbm.at[pl.ds(wid * 128, L)])

    return k(idx_flat, tgt_flat, emb_pad, lse)


def kernel(idx, targets, tok_emb):
    idx_flat = idx.reshape(-1).astype(jnp.int32)
    tgt_flat = targets.reshape(-1).astype(jnp.int32)
    lse, emb_pad = _row_lse(tok_emb)
    logits_pad, partials = _sc_gather(idx_flat, tgt_flat, emb_pad, lse)
    logits = logits_pad[:, :VOCAB].reshape(idx.shape[0], idx.shape[1], VOCAB)
    loss = jnp.sum(partials.reshape(NW, 128)[:, :L]) / N_TOK
    return (logits, loss)
